# baseline (speedup 1.0000x reference)
import functools

import jax
import jax.numpy as jnp
from jax import lax
from jax.experimental import pallas as pl
from jax.experimental.pallas import tpu as pltpu
from jax.experimental.pallas import tpu_sc as plsc

_NC = 2
_NS = 16
_NW = _NC * _NS
_B = 4096 * 50
_D = 128
_L = 16
_BPW = _B // _NW
_C = 400
_NCHUNK = _BPW // _C

_mesh = plsc.VectorSubcoreMesh(core_axis_name="c", subcore_axis_name="s")

_DNUMS = lax.GatherDimensionNumbers(
    offset_dims=(), collapsed_slice_dims=(0,), start_index_map=(0,))


def _bcast_lane(vec, j):
    idx = jnp.full((_L, 1), j, dtype=jnp.int32)
    return lax.gather(vec, idx, _DNUMS, slice_sizes=(1,),
                      mode=lax.GatherScatterMode.PROMISE_IN_BOUNDS)


@functools.partial(
    pl.kernel,
    out_type=jax.ShapeDtypeStruct((_B, _D), jnp.float32),
    mesh=_mesh,
    scratch_types=[
        pltpu.VMEM((2, _D), jnp.float32),
        pltpu.VMEM((_BPW,), jnp.int32),
        pltpu.VMEM((_C, _D), jnp.float32),
        pltpu.VMEM((_C, _D), jnp.float32),
        pltpu.SemaphoreType.DMA,
        pltpu.SemaphoreType.DMA,
    ],
)
def _emb_lookup_sc(idx_hbm, table_hbm, out_hbm,
                   table_v, idx_v, rows_a, rows_b, sem_a, sem_b):
    wid = lax.axis_index("s") * _NC + lax.axis_index("c")
    base = wid * _BPW
    pltpu.sync_copy(table_hbm, table_v)
    pltpu.sync_copy(idx_hbm.at[pl.ds(base, _BPW)], idx_v)
    row1 = [table_v[1, pl.ds(k * _L, _L)] for k in range(_D // _L)]
    diff = [table_v[0, pl.ds(k * _L, _L)] - row1[k] for k in range(_D // _L)]
    bufs = [rows_a, rows_b]
    sems = [sem_a, sem_b]
    copies = [None, None]
    for i in range(_NCHUNK):
        b = i % 2
        buf = bufs[b]
        if copies[b] is not None:
            copies[b].wait()

        def body(g, carry, i=i, buf=buf):
            i16 = idx_v[pl.ds(i * _C + g * _L, _L)]
            mf = (1 - i16).astype(jnp.float32)
            for j in range(_L):
                m = _bcast_lane(mf, j)
                for k in range(_D // _L):
                    buf[g * _L + j, pl.ds(k * _L, _L)] = m * diff[k] + row1[k]
            return carry

        lax.fori_loop(0, _C // _L, body, 0)
        copies[b] = pltpu.async_copy(
            buf, out_hbm.at[pl.ds(base + i * _C, _C)], sems[b])
    copies[0].wait()
    copies[1].wait()


def kernel(inputs, table):
    idx = inputs.reshape(_B)
    out = _emb_lookup_sc(idx, table)
    return out.reshape(inputs.shape[0], inputs.shape[1], _D)


# speedup vs baseline: 2.0338x; 2.0338x over previous
import functools

import jax
import jax.numpy as jnp
from jax import lax
from jax.experimental import pallas as pl
from jax.experimental.pallas import tpu as pltpu
from jax.experimental.pallas import tpu_sc as plsc

_NC = 2
_NS = 16
_NW = _NC * _NS
_R = 4096
_S = 50
_B = _R * _S
_D = 128
_L = 16
_BPW = _B // _NW
_RPW = _R // _NW
_CR = 8
_C = _CR * _S
_NCHUNK = _RPW // _CR

_mesh = plsc.VectorSubcoreMesh(core_axis_name="c", subcore_axis_name="s")

_DNUMS = lax.GatherDimensionNumbers(
    offset_dims=(), collapsed_slice_dims=(0,), start_index_map=(0,))


def _bcast_lane(vec, j):
    idx = jnp.full((_L, 1), j, dtype=jnp.int32)
    return lax.gather(vec, idx, _DNUMS, slice_sizes=(1,),
                      mode=lax.GatherScatterMode.PROMISE_IN_BOUNDS)


@functools.partial(
    pl.kernel,
    out_type=jax.ShapeDtypeStruct((_R, _S, _D), jnp.float32),
    mesh=_mesh,
    compiler_params=pltpu.CompilerParams(use_tc_tiling_on_sc=True),
    scratch_types=[
        pltpu.VMEM((2, _D), jnp.float32),
        pltpu.VMEM((_BPW,), jnp.int32),
        pltpu.VMEM((_CR, _S, _D), jnp.float32),
        pltpu.VMEM((_CR, _S, _D), jnp.float32),
        pltpu.SemaphoreType.DMA,
        pltpu.SemaphoreType.DMA,
    ],
)
def _emb_lookup_sc(idx_hbm, table_hbm, out_hbm,
                   table_v, idx_v, rows_a, rows_b, sem_a, sem_b):
    wid = lax.axis_index("s") * _NC + lax.axis_index("c")
    base = wid * _BPW
    pltpu.sync_copy(table_hbm, table_v)
    pltpu.sync_copy(idx_hbm.at[pl.ds(base, _BPW)], idx_v)
    row1 = [table_v[1, pl.ds(k * _L, _L)] for k in range(_D // _L)]
    diff = [table_v[0, pl.ds(k * _L, _L)] - row1[k] for k in range(_D // _L)]
    bufs = [rows_a, rows_b]
    sems = [sem_a, sem_b]
    copies = [None, None]
    for i in range(_NCHUNK):
        b = i % 2
        buf = bufs[b]
        if copies[b] is not None:
            copies[b].wait()

        def body(g, carry, i=i, buf=buf):
            i16 = idx_v[pl.ds(i * _C + g * _L, _L)]
            mf = (1 - i16).astype(jnp.float32)
            for j in range(_L):
                p = g * _L + j
                r = p // _S
                c = p - r * _S
                m = _bcast_lane(mf, j)
                for k in range(_D // _L):
                    buf[r, c, pl.ds(k * _L, _L)] = m * diff[k] + row1[k]
            return carry

        lax.fori_loop(0, _C // _L, body, 0)
        copies[b] = pltpu.async_copy(
            buf, out_hbm.at[pl.ds(wid * _RPW + i * _CR, _CR)], sems[b])
    copies[0].wait()
    copies[1].wait()


def kernel(inputs, table):
    idx = inputs.reshape(_B)
    return _emb_lookup_sc(idx, table)
